# baseline (speedup 1.0000x reference)
import functools

import jax
import jax.numpy as jnp
from jax import lax
from jax.experimental import pallas as pl
from jax.experimental.pallas import tpu as pltpu
from jax.experimental.pallas import tpu_sc as plsc

DIM = 128
B = 16384
NCOMB = 4096

NC = 2
NS = 16
NW = NC * NS
L = 16
BPW = B // NW
CH = 128
NCHUNK = BPW // CH


def _tables_body(pt, at, ot, gt, ws, wc, bs, bc, t_ref, s_ref, c_ref):
    kk = lax.broadcasted_iota(jnp.int32, (NCOMB, 1), 0)
    ohp = (kk // 2048 == lax.broadcasted_iota(jnp.int32, (NCOMB, 2), 1)).astype(jnp.float32)
    oha = ((kk // 256) % 8 == lax.broadcasted_iota(jnp.int32, (NCOMB, 8), 1)).astype(jnp.float32)
    oho = ((kk // 16) % 16 == lax.broadcasted_iota(jnp.int32, (NCOMB, 16), 1)).astype(jnp.float32)
    ohg = (kk % 16 == lax.broadcasted_iota(jnp.int32, (NCOMB, 16), 1)).astype(jnp.float32)
    t = (jnp.dot(ohp, pt[...], preferred_element_type=jnp.float32, precision=lax.Precision.HIGHEST)
         + jnp.dot(oha, at[...], preferred_element_type=jnp.float32, precision=lax.Precision.HIGHEST)
         + jnp.dot(oho, ot[...], preferred_element_type=jnp.float32, precision=lax.Precision.HIGHEST)
         + jnp.dot(ohg, gt[...], preferred_element_type=jnp.float32, precision=lax.Precision.HIGHEST))
    t_ref[...] = t
    s_ref[...] = jnp.dot(t, ws[...], preferred_element_type=jnp.float32, precision=lax.Precision.HIGHEST) + bs[...]
    c_ref[...] = jnp.dot(t, wc[...], preferred_element_type=jnp.float32, precision=lax.Precision.HIGHEST) + bc[...]


def _build_tables(pt, at, ot, gt, ws, wc, bs, bc):
    return pl.pallas_call(
        _tables_body,
        out_shape=[
            jax.ShapeDtypeStruct((NCOMB, 2 * DIM), jnp.float32),
            jax.ShapeDtypeStruct((NCOMB, 1), jnp.float32),
            jax.ShapeDtypeStruct((NCOMB, 1), jnp.float32),
        ],
    )(pt, at, ot, gt, ws, wc, bs, bc)


def _sc_body(t_h, s_h, c_h, p_h, a_h, o_h, g_h, nz_h,
             out_h, sal_h, conf_h,
             kidx, pv, av, ov, gv, nzv, sv, cv, salv, confv,
             buf0, buf1, gs0, gs1, os0, os1):
    wid = lax.axis_index("s") * NC + lax.axis_index("c")
    base = wid * BPW
    pltpu.sync_copy(p_h.at[pl.ds(base, BPW)], pv)
    pltpu.sync_copy(a_h.at[pl.ds(base, BPW)], av)
    pltpu.sync_copy(o_h.at[pl.ds(base, BPW)], ov)
    pltpu.sync_copy(g_h.at[pl.ds(base, BPW)], gv)
    pltpu.sync_copy(nz_h.at[pl.ds(base, BPW)], nzv)
    pltpu.sync_copy(s_h, sv)
    pltpu.sync_copy(c_h, cv)

    bufs = (buf0, buf1)
    gsems = (gs0, gs1)
    osems = (os0, os1)
    gh = [None, None]
    oh = [None, None]
    for c in range(NCHUNK):
        for j in range(CH // L):
            sl = pl.ds(c * CH + j * L, L)
            kv = ((pv[sl] * 8 + av[sl]) * 16 + ov[sl]) * 16 + gv[sl]
            kidx[c, pl.ds(j * L, L)] = kv
            krow = lax.shift_right_logical(kv, 7)
            kcol = jnp.bitwise_and(kv, 127)
            salv[sl] = plsc.load_gather(sv, [krow, kcol]) + nzv[sl]
            cl = plsc.load_gather(cv, [krow, kcol])
            confv[sl] = 1.0 / (1.0 + jnp.exp(-cl))
        slot = c % 2
        if oh[slot] is not None:
            oh[slot].wait()
        gh[slot] = pltpu.async_copy(t_h.at[kidx.at[c]], bufs[slot], gsems[slot])
        if c >= 1:
            prev = (c - 1) % 2
            gh[prev].wait()
            oh[prev] = pltpu.async_copy(
                bufs[prev], out_h.at[pl.ds(base + (c - 1) * CH, CH)], osems[prev])
    last = (NCHUNK - 1) % 2
    gh[last].wait()
    oh[last] = pltpu.async_copy(
        bufs[last], out_h.at[pl.ds(base + (NCHUNK - 1) * CH, CH)], osems[last])
    pltpu.sync_copy(salv, sal_h.at[pl.ds(base, BPW)])
    pltpu.sync_copy(confv, conf_h.at[pl.ds(base, BPW)])
    for h in oh:
        if h is not None:
            h.wait()


def _sc_lookup(*args):
    return pl.kernel(
        _sc_body,
        out_type=[
            jax.ShapeDtypeStruct((B, 2 * DIM), jnp.float32),
            jax.ShapeDtypeStruct((B,), jnp.float32),
            jax.ShapeDtypeStruct((B,), jnp.float32),
        ],
        mesh=plsc.VectorSubcoreMesh(
            core_axis_name="c", subcore_axis_name="s",
            num_cores=NC, num_subcores=NS),
        compiler_params=pltpu.CompilerParams(needs_layout_passes=False),
        scratch_types=[
        pltpu.VMEM((NCHUNK, CH), jnp.int32),
        pltpu.VMEM((BPW,), jnp.int32),
        pltpu.VMEM((BPW,), jnp.int32),
        pltpu.VMEM((BPW,), jnp.int32),
        pltpu.VMEM((BPW,), jnp.int32),
        pltpu.VMEM((BPW,), jnp.float32),
        pltpu.VMEM((NCOMB // 128, 128), jnp.float32),
        pltpu.VMEM((NCOMB // 128, 128), jnp.float32),
        pltpu.VMEM((BPW,), jnp.float32),
        pltpu.VMEM((BPW,), jnp.float32),
        pltpu.VMEM((CH, 2 * DIM), jnp.float32),
        pltpu.VMEM((CH, 2 * DIM), jnp.float32),
        pltpu.SemaphoreType.DMA,
        pltpu.SemaphoreType.DMA,
        pltpu.SemaphoreType.DMA,
        pltpu.SemaphoreType.DMA,
        ],
    )(*args)


def kernel(p_idx, a_idx, o_idx, g_idx, noise, perspective_emb, audio_dir_emb,
           olfactory_loc_emb, gustatory_loc_emb, W_sal, b_sal, W_conf, b_conf):
    f32 = jnp.float32
    z2 = jnp.zeros((2, DIM), f32)
    z8 = jnp.zeros((8, DIM), f32)
    pt = jnp.stack([perspective_emb.astype(f32), z2], axis=-1).reshape(2, 2 * DIM)
    at = jnp.stack([z8, audio_dir_emb.astype(f32)], axis=-1).reshape(8, 2 * DIM)
    o_ = olfactory_loc_emb.astype(f32)
    g_ = gustatory_loc_emb.astype(f32)
    ot = jnp.stack([o_[:, :DIM], o_[:, DIM:]], axis=-1).reshape(16, 2 * DIM)
    gt = jnp.stack([g_[:, :DIM], g_[:, DIM:]], axis=-1).reshape(16, 2 * DIM)
    ws = jnp.stack([W_sal[:DIM, 0], W_sal[DIM:, 0]], axis=-1).reshape(2 * DIM, 1).astype(f32)
    wc = jnp.stack([W_conf[:DIM, 0], W_conf[DIM:, 0]], axis=-1).reshape(2 * DIM, 1).astype(f32)
    bs = b_sal.reshape(1, 1).astype(f32)
    bc = b_conf.reshape(1, 1).astype(f32)

    t, s, c = _build_tables(pt, at, ot, gt, ws, wc, bs, bc)

    rows, sal, conf = _sc_lookup(
        t, s.reshape(NCOMB // 128, 128), c.reshape(NCOMB // 128, 128),
        p_idx.astype(jnp.int32), a_idx.astype(jnp.int32),
        o_idx.astype(jnp.int32), g_idx.astype(jnp.int32),
        noise.astype(f32).reshape(B))

    proposal = rows.view(jnp.complex64)
    return proposal, sal.reshape(B, 1), conf.reshape(B, 1)


# speedup vs baseline: 3.5128x; 3.5128x over previous
import functools

import jax
import jax.numpy as jnp
from jax import lax
from jax.experimental import pallas as pl
from jax.experimental.pallas import tpu as pltpu
from jax.experimental.pallas import tpu_sc as plsc

DIM = 128
B = 16384
NCOMB = 4096

NC = 2
NS = 16
NW = NC * NS
L = 16
BPW = B // NW
CH = 128
NCHUNK = BPW // CH


def _tables_body(pt, at, ot, gt, ws, wc, bs, bc, t_ref, s_ref, c_ref):
    kk = lax.broadcasted_iota(jnp.int32, (NCOMB, 1), 0)
    ohp = (kk // 2048 == lax.broadcasted_iota(jnp.int32, (NCOMB, 2), 1)).astype(jnp.float32)
    oha = ((kk // 256) % 8 == lax.broadcasted_iota(jnp.int32, (NCOMB, 8), 1)).astype(jnp.float32)
    oho = ((kk // 16) % 16 == lax.broadcasted_iota(jnp.int32, (NCOMB, 16), 1)).astype(jnp.float32)
    ohg = (kk % 16 == lax.broadcasted_iota(jnp.int32, (NCOMB, 16), 1)).astype(jnp.float32)
    t = (jnp.dot(ohp, pt[...], preferred_element_type=jnp.float32, precision=lax.Precision.HIGHEST)
         + jnp.dot(oha, at[...], preferred_element_type=jnp.float32, precision=lax.Precision.HIGHEST)
         + jnp.dot(oho, ot[...], preferred_element_type=jnp.float32, precision=lax.Precision.HIGHEST)
         + jnp.dot(ohg, gt[...], preferred_element_type=jnp.float32, precision=lax.Precision.HIGHEST))
    t_ref[...] = t
    s_ref[...] = jnp.dot(t, ws[...], preferred_element_type=jnp.float32, precision=lax.Precision.HIGHEST) + bs[...]
    c_ref[...] = jnp.dot(t, wc[...], preferred_element_type=jnp.float32, precision=lax.Precision.HIGHEST) + bc[...]


def _build_tables(pt, at, ot, gt, ws, wc, bs, bc):
    return pl.pallas_call(
        _tables_body,
        out_shape=[
            jax.ShapeDtypeStruct((NCOMB, 2 * DIM), jnp.float32),
            jax.ShapeDtypeStruct((NCOMB, 1), jnp.float32),
            jax.ShapeDtypeStruct((NCOMB, 1), jnp.float32),
        ],
    )(pt, at, ot, gt, ws, wc, bs, bc)


def _sc_body(t_h, s_h, c_h, p_h, a_h, o_h, g_h, nz_h,
             out_h, sal_h, conf_h,
             kidx, pv, av, ov, gv, nzv, sv, cv, salv, confv,
             buf0, buf1, gs0, gs1, os0, os1):
    wid = lax.axis_index("s") * NC + lax.axis_index("c")
    base = wid * BPW
    pltpu.sync_copy(p_h.at[pl.ds(base, BPW)], pv)
    pltpu.sync_copy(a_h.at[pl.ds(base, BPW)], av)
    pltpu.sync_copy(o_h.at[pl.ds(base, BPW)], ov)
    pltpu.sync_copy(g_h.at[pl.ds(base, BPW)], gv)
    pltpu.sync_copy(nz_h.at[pl.ds(base, BPW)], nzv)
    pltpu.sync_copy(s_h, sv)
    pltpu.sync_copy(c_h, cv)

    bufs = (buf0, buf1)
    gsems = (gs0, gs1)
    osems = (os0, os1)
    gh = [None, None]
    oh = [None, None]
    for c in range(NCHUNK):
        for j in range(CH // L):
            sl = pl.ds(c * CH + j * L, L)
            kv = ((pv[sl] * 8 + av[sl]) * 16 + ov[sl]) * 16 + gv[sl]
            kidx[c, pl.ds(j * L, L)] = kv
            krow = lax.shift_right_logical(kv, 7)
            kcol = jnp.bitwise_and(kv, 127)
            salv[sl] = plsc.load_gather(sv, [krow, kcol]) + nzv[sl]
            cl = plsc.load_gather(cv, [krow, kcol])
            confv[sl] = 1.0 / (1.0 + jnp.exp(-cl))
        slot = c % 2
        if oh[slot] is not None:
            oh[slot].wait()
        gh[slot] = pltpu.async_copy(t_h.at[kidx.at[c]], bufs[slot], gsems[slot])
        if c >= 1:
            prev = (c - 1) % 2
            gh[prev].wait()
            oh[prev] = pltpu.async_copy(
                bufs[prev], out_h.at[pl.ds(base + (c - 1) * CH, CH)], osems[prev])
    last = (NCHUNK - 1) % 2
    gh[last].wait()
    oh[last] = pltpu.async_copy(
        bufs[last], out_h.at[pl.ds(base + (NCHUNK - 1) * CH, CH)], osems[last])
    pltpu.sync_copy(salv, sal_h.at[pl.ds(base, BPW)])
    pltpu.sync_copy(confv, conf_h.at[pl.ds(base, BPW)])
    for h in oh:
        if h is not None:
            h.wait()


def _sc_lookup(*args):
    return pl.kernel(
        _sc_body,
        out_type=[
            jax.ShapeDtypeStruct((B, 2 * DIM), jnp.float32),
            jax.ShapeDtypeStruct((B,), jnp.float32),
            jax.ShapeDtypeStruct((B,), jnp.float32),
        ],
        mesh=plsc.VectorSubcoreMesh(
            core_axis_name="c", subcore_axis_name="s",
            num_cores=NC, num_subcores=NS),
        compiler_params=pltpu.CompilerParams(needs_layout_passes=False),
        scratch_types=[
        pltpu.VMEM((NCHUNK, CH), jnp.int32),
        pltpu.VMEM((BPW,), jnp.int32),
        pltpu.VMEM((BPW,), jnp.int32),
        pltpu.VMEM((BPW,), jnp.int32),
        pltpu.VMEM((BPW,), jnp.int32),
        pltpu.VMEM((BPW,), jnp.float32),
        pltpu.VMEM((NCOMB // 128, 128), jnp.float32),
        pltpu.VMEM((NCOMB // 128, 128), jnp.float32),
        pltpu.VMEM((BPW,), jnp.float32),
        pltpu.VMEM((BPW,), jnp.float32),
        pltpu.VMEM((CH, 2 * DIM), jnp.float32),
        pltpu.VMEM((CH, 2 * DIM), jnp.float32),
        pltpu.SemaphoreType.DMA,
        pltpu.SemaphoreType.DMA,
        pltpu.SemaphoreType.DMA,
        pltpu.SemaphoreType.DMA,
        ],
    )(*args)


def kernel(p_idx, a_idx, o_idx, g_idx, noise, perspective_emb, audio_dir_emb,
           olfactory_loc_emb, gustatory_loc_emb, W_sal, b_sal, W_conf, b_conf):
    f32 = jnp.float32
    z2 = jnp.zeros((2, DIM), f32)
    z8 = jnp.zeros((8, DIM), f32)
    pt = jnp.concatenate([perspective_emb.astype(f32), z2], axis=1)
    at = jnp.concatenate([z8, audio_dir_emb.astype(f32)], axis=1)
    ot = olfactory_loc_emb.astype(f32)
    gt = gustatory_loc_emb.astype(f32)
    ws = W_sal.astype(f32)
    wc = W_conf.astype(f32)
    bs = b_sal.reshape(1, 1).astype(f32)
    bc = b_conf.reshape(1, 1).astype(f32)

    t, s, c = _build_tables(pt, at, ot, gt, ws, wc, bs, bc)

    rows, sal, conf = _sc_lookup(
        t, s.reshape(NCOMB // 128, 128), c.reshape(NCOMB // 128, 128),
        p_idx.astype(jnp.int32), a_idx.astype(jnp.int32),
        o_idx.astype(jnp.int32), g_idx.astype(jnp.int32),
        noise.astype(f32).reshape(B))

    proposal = lax.complex(rows[:, :DIM], rows[:, DIM:])
    return proposal, sal.reshape(B, 1), conf.reshape(B, 1)


# speedup vs baseline: 3.7153x; 1.0577x over previous
import jax
import jax.numpy as jnp
from jax import lax
from jax.experimental import pallas as pl
from jax.experimental.pallas import tpu as pltpu
from jax.experimental.pallas import tpu_sc as plsc

DIM = 128
B = 16384
NCOMB = 4096

NC = 2
NS = 16
NW = NC * NS
L = 16
BPW = B // NW
CH = 128
NCHUNK = BPW // CH

_HI = lax.Precision.HIGHEST


def _tables_body(pt, at, ot, gt, ws, wc, bs, bc, tr_ref, ti_ref, s_ref, c_ref):
    kk = lax.broadcasted_iota(jnp.int32, (NCOMB, 1), 0)
    ohp = (kk // 2048 == lax.broadcasted_iota(jnp.int32, (NCOMB, 2), 1)).astype(jnp.float32)
    oha = ((kk // 256) % 8 == lax.broadcasted_iota(jnp.int32, (NCOMB, 8), 1)).astype(jnp.float32)
    oho = ((kk // 16) % 16 == lax.broadcasted_iota(jnp.int32, (NCOMB, 16), 1)).astype(jnp.float32)
    ohg = (kk % 16 == lax.broadcasted_iota(jnp.int32, (NCOMB, 16), 1)).astype(jnp.float32)
    t = (jnp.dot(ohp, pt[...], preferred_element_type=jnp.float32, precision=_HI)
         + jnp.dot(oha, at[...], preferred_element_type=jnp.float32, precision=_HI)
         + jnp.dot(oho, ot[...], preferred_element_type=jnp.float32, precision=_HI)
         + jnp.dot(ohg, gt[...], preferred_element_type=jnp.float32, precision=_HI))
    tr_ref[...] = t[:, :DIM]
    ti_ref[...] = t[:, DIM:]
    s_ref[...] = jnp.dot(t, ws[...], preferred_element_type=jnp.float32, precision=_HI) + bs[...]
    c_ref[...] = jnp.dot(t, wc[...], preferred_element_type=jnp.float32, precision=_HI) + bc[...]


def _build_tables(pt, at, ot, gt, ws, wc, bs, bc):
    return pl.pallas_call(
        _tables_body,
        out_shape=[
            jax.ShapeDtypeStruct((NCOMB, DIM), jnp.float32),
            jax.ShapeDtypeStruct((NCOMB, DIM), jnp.float32),
            jax.ShapeDtypeStruct((NCOMB, 1), jnp.float32),
            jax.ShapeDtypeStruct((NCOMB, 1), jnp.float32),
        ],
    )(pt, at, ot, gt, ws, wc, bs, bc)


def _sc_body(tr_h, ti_h, s_h, c_h, p_h, a_h, o_h, g_h, nz_h,
             re_h, im_h, sal_h, conf_h,
             kidx, pv, av, ov, gv, nzv, sv, cv, salv, confv,
             bufr0, bufr1, bufi0, bufi1,
             grs0, grs1, gis0, gis1, ors0, ors1, ois0, ois1):
    wid = lax.axis_index("s") * NC + lax.axis_index("c")
    base = wid * BPW
    pltpu.sync_copy(p_h.at[pl.ds(base, BPW)], pv)
    pltpu.sync_copy(a_h.at[pl.ds(base, BPW)], av)
    pltpu.sync_copy(o_h.at[pl.ds(base, BPW)], ov)
    pltpu.sync_copy(g_h.at[pl.ds(base, BPW)], gv)
    pltpu.sync_copy(nz_h.at[pl.ds(base, BPW)], nzv)
    pltpu.sync_copy(s_h, sv)
    pltpu.sync_copy(c_h, cv)

    bufr = (bufr0, bufr1)
    bufi = (bufi0, bufi1)
    grs = (grs0, grs1)
    gis = (gis0, gis1)
    ors = (ors0, ors1)
    ois = (ois0, ois1)
    ghr = [None, None]
    ghi = [None, None]
    ohr = [None, None]
    ohi = [None, None]
    for c in range(NCHUNK):
        for j in range(CH // L):
            sl = pl.ds(c * CH + j * L, L)
            kv = ((pv[sl] * 8 + av[sl]) * 16 + ov[sl]) * 16 + gv[sl]
            kidx[c, pl.ds(j * L, L)] = kv
            krow = lax.shift_right_logical(kv, 7)
            kcol = jnp.bitwise_and(kv, 127)
            salv[sl] = plsc.load_gather(sv, [krow, kcol]) + nzv[sl]
            cl = plsc.load_gather(cv, [krow, kcol])
            confv[sl] = 1.0 / (1.0 + jnp.exp(-cl))
        slot = c % 2
        if ohr[slot] is not None:
            ohr[slot].wait()
            ohi[slot].wait()
        ghr[slot] = pltpu.async_copy(tr_h.at[kidx.at[c]], bufr[slot], grs[slot])
        ghi[slot] = pltpu.async_copy(ti_h.at[kidx.at[c]], bufi[slot], gis[slot])
        if c >= 1:
            prev = (c - 1) % 2
            dst = pl.ds(base + (c - 1) * CH, CH)
            ghr[prev].wait()
            ohr[prev] = pltpu.async_copy(bufr[prev], re_h.at[dst], ors[prev])
            ghi[prev].wait()
            ohi[prev] = pltpu.async_copy(bufi[prev], im_h.at[dst], ois[prev])
    last = (NCHUNK - 1) % 2
    dst = pl.ds(base + (NCHUNK - 1) * CH, CH)
    ghr[last].wait()
    ohr[last] = pltpu.async_copy(bufr[last], re_h.at[dst], ors[last])
    ghi[last].wait()
    ohi[last] = pltpu.async_copy(bufi[last], im_h.at[dst], ois[last])
    pltpu.sync_copy(salv, sal_h.at[pl.ds(base, BPW)])
    pltpu.sync_copy(confv, conf_h.at[pl.ds(base, BPW)])
    for h in ohr + ohi:
        if h is not None:
            h.wait()


def _sc_lookup(*args):
    return pl.kernel(
        _sc_body,
        out_type=[
            jax.ShapeDtypeStruct((B, DIM), jnp.float32),
            jax.ShapeDtypeStruct((B, DIM), jnp.float32),
            jax.ShapeDtypeStruct((B,), jnp.float32),
            jax.ShapeDtypeStruct((B,), jnp.float32),
        ],
        mesh=plsc.VectorSubcoreMesh(
            core_axis_name="c", subcore_axis_name="s",
            num_cores=NC, num_subcores=NS),
        compiler_params=pltpu.CompilerParams(needs_layout_passes=False),
        scratch_types=[
            pltpu.VMEM((NCHUNK, CH), jnp.int32),
            pltpu.VMEM((BPW,), jnp.int32),
            pltpu.VMEM((BPW,), jnp.int32),
            pltpu.VMEM((BPW,), jnp.int32),
            pltpu.VMEM((BPW,), jnp.int32),
            pltpu.VMEM((BPW,), jnp.float32),
            pltpu.VMEM((NCOMB // 128, 128), jnp.float32),
            pltpu.VMEM((NCOMB // 128, 128), jnp.float32),
            pltpu.VMEM((BPW,), jnp.float32),
            pltpu.VMEM((BPW,), jnp.float32),
            pltpu.VMEM((CH, DIM), jnp.float32),
            pltpu.VMEM((CH, DIM), jnp.float32),
            pltpu.VMEM((CH, DIM), jnp.float32),
            pltpu.VMEM((CH, DIM), jnp.float32),
            pltpu.SemaphoreType.DMA,
            pltpu.SemaphoreType.DMA,
            pltpu.SemaphoreType.DMA,
            pltpu.SemaphoreType.DMA,
            pltpu.SemaphoreType.DMA,
            pltpu.SemaphoreType.DMA,
            pltpu.SemaphoreType.DMA,
            pltpu.SemaphoreType.DMA,
        ],
    )(*args)


def kernel(p_idx, a_idx, o_idx, g_idx, noise, perspective_emb, audio_dir_emb,
           olfactory_loc_emb, gustatory_loc_emb, W_sal, b_sal, W_conf, b_conf):
    f32 = jnp.float32
    z2 = jnp.zeros((2, DIM), f32)
    z8 = jnp.zeros((8, DIM), f32)
    pt = jnp.concatenate([perspective_emb.astype(f32), z2], axis=1)
    at = jnp.concatenate([z8, audio_dir_emb.astype(f32)], axis=1)
    ot = olfactory_loc_emb.astype(f32)
    gt = gustatory_loc_emb.astype(f32)
    ws = W_sal.astype(f32)
    wc = W_conf.astype(f32)
    bs = b_sal.reshape(1, 1).astype(f32)
    bc = b_conf.reshape(1, 1).astype(f32)

    tr, ti, s, c = _build_tables(pt, at, ot, gt, ws, wc, bs, bc)

    re, im, sal, conf = _sc_lookup(
        tr, ti, s.reshape(NCOMB // 128, 128), c.reshape(NCOMB // 128, 128),
        p_idx.astype(jnp.int32), a_idx.astype(jnp.int32),
        o_idx.astype(jnp.int32), g_idx.astype(jnp.int32),
        noise.astype(f32).reshape(B))

    proposal = lax.complex(re, im)
    return proposal, sal.reshape(B, 1), conf.reshape(B, 1)


# speedup vs baseline: 4.0394x; 1.0872x over previous
import jax
import jax.numpy as jnp
from jax import lax
from jax.experimental import pallas as pl
from jax.experimental.pallas import tpu as pltpu
from jax.experimental.pallas import tpu_sc as plsc

DIM = 128
B = 16384
NCOMB = 4096

NC = 2
NS = 16
NW = NC * NS
L = 16
BPW = B // NW
CH = 128
NCHUNK = BPW // CH

_HI = lax.Precision.HIGHEST


def _tables_body(p_t, a_t, o_t, g_t, ws, wc, bs, bc, tr_ref, ti_ref, s_ref, c_ref):
    def dot(x, y):
        return jnp.dot(x, y, preferred_element_type=jnp.float32, precision=_HI)
    kk = lax.broadcasted_iota(jnp.int32, (NCOMB, 1), 0)
    ohp = (kk // 2048 == lax.broadcasted_iota(jnp.int32, (NCOMB, 2), 1)).astype(jnp.float32)
    oha = ((kk // 256) % 8 == lax.broadcasted_iota(jnp.int32, (NCOMB, 8), 1)).astype(jnp.float32)
    oho = ((kk // 16) % 16 == lax.broadcasted_iota(jnp.int32, (NCOMB, 16), 1)).astype(jnp.float32)
    ohg = (kk % 16 == lax.broadcasted_iota(jnp.int32, (NCOMB, 16), 1)).astype(jnp.float32)
    o_ = o_t[...]
    g_ = g_t[...]
    tr = dot(ohp, p_t[...]) + dot(oho, o_[:, :DIM]) + dot(ohg, g_[:, :DIM])
    ti = dot(oha, a_t[...]) + dot(oho, o_[:, DIM:]) + dot(ohg, g_[:, DIM:])
    tr_ref[...] = tr
    ti_ref[...] = ti
    w_s = ws[...]
    w_c = wc[...]
    s_ref[...] = dot(tr, w_s[:DIM]) + dot(ti, w_s[DIM:]) + bs[...]
    c_ref[...] = dot(tr, w_c[:DIM]) + dot(ti, w_c[DIM:]) + bc[...]


def _build_tables(p_t, a_t, o_t, g_t, ws, wc, bs, bc):
    return pl.pallas_call(
        _tables_body,
        out_shape=[
            jax.ShapeDtypeStruct((NCOMB, DIM), jnp.float32),
            jax.ShapeDtypeStruct((NCOMB, DIM), jnp.float32),
            jax.ShapeDtypeStruct((NCOMB, 1), jnp.float32),
            jax.ShapeDtypeStruct((NCOMB, 1), jnp.float32),
        ],
    )(p_t, a_t, o_t, g_t, ws, wc, bs, bc)


def _sc_body(tr_h, ti_h, s_h, c_h, p_h, a_h, o_h, g_h, nz_h,
             re_h, im_h, sal_h, conf_h,
             kidx, pv, av, ov, gv, nzv, sv, cv, salv, confv,
             bufr0, bufr1, bufi0, bufi1,
             grs0, grs1, gis0, gis1, ors0, ors1, ois0, ois1):
    wid = lax.axis_index("s") * NC + lax.axis_index("c")
    base = wid * BPW
    pltpu.sync_copy(p_h.at[pl.ds(base, BPW)], pv)
    pltpu.sync_copy(a_h.at[pl.ds(base, BPW)], av)
    pltpu.sync_copy(o_h.at[pl.ds(base, BPW)], ov)
    pltpu.sync_copy(g_h.at[pl.ds(base, BPW)], gv)
    pltpu.sync_copy(nz_h.at[pl.ds(base, BPW)], nzv)
    pltpu.sync_copy(s_h, sv)
    pltpu.sync_copy(c_h, cv)

    bufr = (bufr0, bufr1)
    bufi = (bufi0, bufi1)
    grs = (grs0, grs1)
    gis = (gis0, gis1)
    ors = (ors0, ors1)
    ois = (ois0, ois1)
    ghr = [None, None]
    ghi = [None, None]
    ohr = [None, None]
    ohi = [None, None]
    for c in range(NCHUNK):
        for j in range(CH // L):
            sl = pl.ds(c * CH + j * L, L)
            kv = ((pv[sl] * 8 + av[sl]) * 16 + ov[sl]) * 16 + gv[sl]
            kidx[c, pl.ds(j * L, L)] = kv
            krow = lax.shift_right_logical(kv, 7)
            kcol = jnp.bitwise_and(kv, 127)
            salv[sl] = plsc.load_gather(sv, [krow, kcol]) + nzv[sl]
            cl = plsc.load_gather(cv, [krow, kcol])
            confv[sl] = 1.0 / (1.0 + jnp.exp(-cl))
        slot = c % 2
        if ohr[slot] is not None:
            ohr[slot].wait()
            ohi[slot].wait()
        ghr[slot] = pltpu.async_copy(tr_h.at[kidx.at[c]], bufr[slot], grs[slot])
        ghi[slot] = pltpu.async_copy(ti_h.at[kidx.at[c]], bufi[slot], gis[slot])
        if c >= 1:
            prev = (c - 1) % 2
            dst = pl.ds(base + (c - 1) * CH, CH)
            ghr[prev].wait()
            ohr[prev] = pltpu.async_copy(bufr[prev], re_h.at[dst], ors[prev])
            ghi[prev].wait()
            ohi[prev] = pltpu.async_copy(bufi[prev], im_h.at[dst], ois[prev])
    last = (NCHUNK - 1) % 2
    dst = pl.ds(base + (NCHUNK - 1) * CH, CH)
    ghr[last].wait()
    ohr[last] = pltpu.async_copy(bufr[last], re_h.at[dst], ors[last])
    ghi[last].wait()
    ohi[last] = pltpu.async_copy(bufi[last], im_h.at[dst], ois[last])
    pltpu.sync_copy(salv, sal_h.at[pl.ds(base, BPW)])
    pltpu.sync_copy(confv, conf_h.at[pl.ds(base, BPW)])
    for h in ohr + ohi:
        if h is not None:
            h.wait()


def _sc_lookup(*args):
    return pl.kernel(
        _sc_body,
        out_type=[
            jax.ShapeDtypeStruct((B, DIM), jnp.float32),
            jax.ShapeDtypeStruct((B, DIM), jnp.float32),
            jax.ShapeDtypeStruct((B,), jnp.float32),
            jax.ShapeDtypeStruct((B,), jnp.float32),
        ],
        mesh=plsc.VectorSubcoreMesh(
            core_axis_name="c", subcore_axis_name="s",
            num_cores=NC, num_subcores=NS),
        compiler_params=pltpu.CompilerParams(needs_layout_passes=False),
        scratch_types=[
            pltpu.VMEM((NCHUNK, CH), jnp.int32),
            pltpu.VMEM((BPW,), jnp.int32),
            pltpu.VMEM((BPW,), jnp.int32),
            pltpu.VMEM((BPW,), jnp.int32),
            pltpu.VMEM((BPW,), jnp.int32),
            pltpu.VMEM((BPW,), jnp.float32),
            pltpu.VMEM((NCOMB // 128, 128), jnp.float32),
            pltpu.VMEM((NCOMB // 128, 128), jnp.float32),
            pltpu.VMEM((BPW,), jnp.float32),
            pltpu.VMEM((BPW,), jnp.float32),
            pltpu.VMEM((CH, DIM), jnp.float32),
            pltpu.VMEM((CH, DIM), jnp.float32),
            pltpu.VMEM((CH, DIM), jnp.float32),
            pltpu.VMEM((CH, DIM), jnp.float32),
            pltpu.SemaphoreType.DMA,
            pltpu.SemaphoreType.DMA,
            pltpu.SemaphoreType.DMA,
            pltpu.SemaphoreType.DMA,
            pltpu.SemaphoreType.DMA,
            pltpu.SemaphoreType.DMA,
            pltpu.SemaphoreType.DMA,
            pltpu.SemaphoreType.DMA,
        ],
    )(*args)


def kernel(p_idx, a_idx, o_idx, g_idx, noise, perspective_emb, audio_dir_emb,
           olfactory_loc_emb, gustatory_loc_emb, W_sal, b_sal, W_conf, b_conf):
    f32 = jnp.float32
    bs = b_sal.reshape(1, 1).astype(f32)
    bc = b_conf.reshape(1, 1).astype(f32)

    tr, ti, s, c = _build_tables(
        perspective_emb.astype(f32), audio_dir_emb.astype(f32),
        olfactory_loc_emb.astype(f32), gustatory_loc_emb.astype(f32),
        W_sal.astype(f32), W_conf.astype(f32), bs, bc)

    re, im, sal, conf = _sc_lookup(
        tr, ti, s.reshape(NCOMB // 128, 128), c.reshape(NCOMB // 128, 128),
        p_idx.astype(jnp.int32), a_idx.astype(jnp.int32),
        o_idx.astype(jnp.int32), g_idx.astype(jnp.int32),
        noise.astype(f32).reshape(B))

    proposal = lax.complex(re, im)
    return proposal, sal.reshape(B, 1), conf.reshape(B, 1)


# speedup vs baseline: 4.0777x; 1.0095x over previous
import jax
import jax.numpy as jnp
from jax import lax
from jax.experimental import pallas as pl
from jax.experimental.pallas import tpu as pltpu
from jax.experimental.pallas import tpu_sc as plsc

DIM = 128
B = 16384
NCOMB = 4096

NC = 2
NS = 16
NW = NC * NS
L = 16
BPW = B // NW
CH = 128
NCHUNK = BPW // CH

_HI = lax.Precision.HIGHEST


def _tables_body(p_t, a_t, o_t, g_t, ws, wc, bs, bc, tr_ref, ti_ref, s_ref, c_ref):
    def dot(x, y):
        return jnp.dot(x, y, preferred_element_type=jnp.float32, precision=_HI)
    kk = lax.broadcasted_iota(jnp.int32, (NCOMB, 1), 0)
    ohp = (kk // 2048 == lax.broadcasted_iota(jnp.int32, (NCOMB, 2), 1)).astype(jnp.float32)
    oha = ((kk // 256) % 8 == lax.broadcasted_iota(jnp.int32, (NCOMB, 8), 1)).astype(jnp.float32)
    oho = ((kk // 16) % 16 == lax.broadcasted_iota(jnp.int32, (NCOMB, 16), 1)).astype(jnp.float32)
    ohg = (kk % 16 == lax.broadcasted_iota(jnp.int32, (NCOMB, 16), 1)).astype(jnp.float32)
    o_ = o_t[...]
    g_ = g_t[...]
    tr = dot(ohp, p_t[...]) + dot(oho, o_[:, :DIM]) + dot(ohg, g_[:, :DIM])
    ti = dot(oha, a_t[...]) + dot(oho, o_[:, DIM:]) + dot(ohg, g_[:, DIM:])
    tr_ref[...] = tr
    ti_ref[...] = ti
    w_s = ws[...]
    w_c = wc[...]
    s_ref[...] = dot(tr, w_s[:DIM]) + dot(ti, w_s[DIM:]) + bs[...]
    c_ref[...] = dot(tr, w_c[:DIM]) + dot(ti, w_c[DIM:]) + bc[...]


def _build_tables(p_t, a_t, o_t, g_t, ws, wc, bs, bc):
    return pl.pallas_call(
        _tables_body,
        out_shape=[
            jax.ShapeDtypeStruct((NCOMB, DIM), jnp.float32),
            jax.ShapeDtypeStruct((NCOMB, DIM), jnp.float32),
            jax.ShapeDtypeStruct((NCOMB, 1), jnp.float32),
            jax.ShapeDtypeStruct((NCOMB, 1), jnp.float32),
        ],
    )(p_t, a_t, o_t, g_t, ws, wc, bs, bc)


def _sc_body(tr_h, ti_h, s_h, c_h, p_h, a_h, o_h, g_h, nz_h,
             re_h, im_h, sal_h, conf_h,
             kidx, pv, av, ov, gv, nzv, sv, cv, salv, confv,
             bufr0, bufr1, bufi0, bufi1,
             grs0, grs1, gis0, gis1, ors0, ors1, ois0, ois1):
    wid = lax.axis_index("s") * NC + lax.axis_index("c")
    base = wid * BPW
    sl_in = pl.ds(base, BPW)
    idx_cp = [
        pltpu.async_copy(p_h.at[sl_in], pv, grs0),
        pltpu.async_copy(a_h.at[sl_in], av, grs0),
        pltpu.async_copy(o_h.at[sl_in], ov, grs0),
        pltpu.async_copy(g_h.at[sl_in], gv, grs0),
    ]
    aux_cp = [
        pltpu.async_copy(nz_h.at[sl_in], nzv, gis0),
        pltpu.async_copy(s_h, sv, gis0),
        pltpu.async_copy(c_h, cv, gis0),
    ]
    for h in idx_cp:
        h.wait()

    bufr = (bufr0, bufr1)
    bufi = (bufi0, bufi1)
    grs = (grs0, grs1)
    gis = (gis0, gis1)
    ors = (ors0, ors1)
    ois = (ois0, ois1)
    ghr = [None, None]
    ghi = [None, None]
    ohr = [None, None]
    ohi = [None, None]
    for c in range(NCHUNK):
        for j in range(CH // L):
            sl = pl.ds(c * CH + j * L, L)
            kv = ((pv[sl] * 8 + av[sl]) * 16 + ov[sl]) * 16 + gv[sl]
            kidx[c, pl.ds(j * L, L)] = kv
        slot = c % 2
        if ohr[slot] is not None:
            ohr[slot].wait()
            ohi[slot].wait()
        ghr[slot] = pltpu.async_copy(tr_h.at[kidx.at[c]], bufr[slot], grs[slot])
        ghi[slot] = pltpu.async_copy(ti_h.at[kidx.at[c]], bufi[slot], gis[slot])
        if c >= 1:
            prev = (c - 1) % 2
            dst = pl.ds(base + (c - 1) * CH, CH)
            ghr[prev].wait()
            ohr[prev] = pltpu.async_copy(bufr[prev], re_h.at[dst], ors[prev])
            ghi[prev].wait()
            ohi[prev] = pltpu.async_copy(bufi[prev], im_h.at[dst], ois[prev])
    last = (NCHUNK - 1) % 2
    dst = pl.ds(base + (NCHUNK - 1) * CH, CH)
    ghr[last].wait()
    ohr[last] = pltpu.async_copy(bufr[last], re_h.at[dst], ors[last])
    ghi[last].wait()
    ohi[last] = pltpu.async_copy(bufi[last], im_h.at[dst], ois[last])
    for h in aux_cp:
        h.wait()
    for c in range(NCHUNK):
        for j in range(CH // L):
            sl = pl.ds(c * CH + j * L, L)
            kv = kidx[c, pl.ds(j * L, L)]
            krow = lax.shift_right_logical(kv, 7)
            kcol = jnp.bitwise_and(kv, 127)
            salv[sl] = plsc.load_gather(sv, [krow, kcol]) + nzv[sl]
            cl = plsc.load_gather(cv, [krow, kcol])
            confv[sl] = 1.0 / (1.0 + jnp.exp(-cl))
    pltpu.sync_copy(salv, sal_h.at[pl.ds(base, BPW)])
    pltpu.sync_copy(confv, conf_h.at[pl.ds(base, BPW)])
    for h in ohr + ohi:
        if h is not None:
            h.wait()


def _sc_lookup(*args):
    return pl.kernel(
        _sc_body,
        out_type=[
            jax.ShapeDtypeStruct((B, DIM), jnp.float32),
            jax.ShapeDtypeStruct((B, DIM), jnp.float32),
            jax.ShapeDtypeStruct((B,), jnp.float32),
            jax.ShapeDtypeStruct((B,), jnp.float32),
        ],
        mesh=plsc.VectorSubcoreMesh(
            core_axis_name="c", subcore_axis_name="s",
            num_cores=NC, num_subcores=NS),
        compiler_params=pltpu.CompilerParams(needs_layout_passes=False),
        scratch_types=[
            pltpu.VMEM((NCHUNK, CH), jnp.int32),
            pltpu.VMEM((BPW,), jnp.int32),
            pltpu.VMEM((BPW,), jnp.int32),
            pltpu.VMEM((BPW,), jnp.int32),
            pltpu.VMEM((BPW,), jnp.int32),
            pltpu.VMEM((BPW,), jnp.float32),
            pltpu.VMEM((NCOMB // 128, 128), jnp.float32),
            pltpu.VMEM((NCOMB // 128, 128), jnp.float32),
            pltpu.VMEM((BPW,), jnp.float32),
            pltpu.VMEM((BPW,), jnp.float32),
            pltpu.VMEM((CH, DIM), jnp.float32),
            pltpu.VMEM((CH, DIM), jnp.float32),
            pltpu.VMEM((CH, DIM), jnp.float32),
            pltpu.VMEM((CH, DIM), jnp.float32),
            pltpu.SemaphoreType.DMA,
            pltpu.SemaphoreType.DMA,
            pltpu.SemaphoreType.DMA,
            pltpu.SemaphoreType.DMA,
            pltpu.SemaphoreType.DMA,
            pltpu.SemaphoreType.DMA,
            pltpu.SemaphoreType.DMA,
            pltpu.SemaphoreType.DMA,
        ],
    )(*args)


def kernel(p_idx, a_idx, o_idx, g_idx, noise, perspective_emb, audio_dir_emb,
           olfactory_loc_emb, gustatory_loc_emb, W_sal, b_sal, W_conf, b_conf):
    f32 = jnp.float32
    bs = b_sal.reshape(1, 1).astype(f32)
    bc = b_conf.reshape(1, 1).astype(f32)

    tr, ti, s, c = _build_tables(
        perspective_emb.astype(f32), audio_dir_emb.astype(f32),
        olfactory_loc_emb.astype(f32), gustatory_loc_emb.astype(f32),
        W_sal.astype(f32), W_conf.astype(f32), bs, bc)

    re, im, sal, conf = _sc_lookup(
        tr, ti, s.reshape(NCOMB // 128, 128), c.reshape(NCOMB // 128, 128),
        p_idx.astype(jnp.int32), a_idx.astype(jnp.int32),
        o_idx.astype(jnp.int32), g_idx.astype(jnp.int32),
        noise.astype(f32).reshape(B))

    proposal = lax.complex(re, im)
    return proposal, sal.reshape(B, 1), conf.reshape(B, 1)


# speedup vs baseline: 4.1861x; 1.0266x over previous
import jax
import jax.numpy as jnp
from jax import lax
from jax.experimental import pallas as pl
from jax.experimental.pallas import tpu as pltpu
from jax.experimental.pallas import tpu_sc as plsc

DIM = 128
B = 16384
NCOMB = 4096

NC = 2
NS = 16
NW = NC * NS
L = 16
BPW = B // NW
CH = 128
NCHUNK = BPW // CH

_HI = lax.Precision.HIGHEST


def _tables_body(p_t, a_t, o_t, g_t, ws, wc, bs, bc, tr_ref, ti_ref, s_ref, c_ref):
    def dot(x, y):
        return jnp.dot(x, y, preferred_element_type=jnp.float32, precision=_HI)
    kk = lax.broadcasted_iota(jnp.int32, (NCOMB, 1), 0)
    ohp = (kk // 2048 == lax.broadcasted_iota(jnp.int32, (NCOMB, 2), 1)).astype(jnp.float32)
    oha = ((kk // 256) % 8 == lax.broadcasted_iota(jnp.int32, (NCOMB, 8), 1)).astype(jnp.float32)
    oho = ((kk // 16) % 16 == lax.broadcasted_iota(jnp.int32, (NCOMB, 16), 1)).astype(jnp.float32)
    ohg = (kk % 16 == lax.broadcasted_iota(jnp.int32, (NCOMB, 16), 1)).astype(jnp.float32)
    o_ = o_t[...]
    g_ = g_t[...]
    tr = dot(ohp, p_t[...]) + dot(oho, o_[:, :DIM]) + dot(ohg, g_[:, :DIM])
    ti = dot(oha, a_t[...]) + dot(oho, o_[:, DIM:]) + dot(ohg, g_[:, DIM:])
    tr_ref[...] = tr
    ti_ref[...] = ti
    w_s = ws[...]
    w_c = wc[...]
    s = dot(tr, w_s[:DIM]) + dot(ti, w_s[DIM:]) + bs[...]
    c = dot(tr, w_c[:DIM]) + dot(ti, w_c[DIM:]) + bc[...]
    s_ref[...] = s.reshape(NCOMB // 128, 128)
    c_ref[...] = c.reshape(NCOMB // 128, 128)


def _build_tables(p_t, a_t, o_t, g_t, ws, wc, bs, bc):
    return pl.pallas_call(
        _tables_body,
        out_shape=[
            jax.ShapeDtypeStruct((NCOMB, DIM), jnp.float32),
            jax.ShapeDtypeStruct((NCOMB, DIM), jnp.float32),
            jax.ShapeDtypeStruct((NCOMB // 128, 128), jnp.float32),
            jax.ShapeDtypeStruct((NCOMB // 128, 128), jnp.float32),
        ],
    )(p_t, a_t, o_t, g_t, ws, wc, bs, bc)


def _sc_body(tr_h, ti_h, s_h, c_h, p_h, a_h, o_h, g_h, nz_h,
             re_h, im_h, sal_h, conf_h,
             kidx, pv, av, ov, gv, nzv, sv, cv, salv, confv,
             bufr0, bufr1, bufi0, bufi1,
             grs0, grs1, gis0, gis1, ors0, ors1, ois0, ois1):
    wid = lax.axis_index("s") * NC + lax.axis_index("c")
    base = wid * BPW
    sl_in = pl.ds(base, BPW)
    idx_cp = [
        pltpu.async_copy(p_h.at[sl_in], pv, grs0),
        pltpu.async_copy(a_h.at[sl_in], av, grs0),
        pltpu.async_copy(o_h.at[sl_in], ov, grs0),
        pltpu.async_copy(g_h.at[sl_in], gv, grs0),
    ]
    aux_cp = [
        pltpu.async_copy(nz_h.at[sl_in], nzv, gis0),
        pltpu.async_copy(s_h, sv, gis0),
        pltpu.async_copy(c_h, cv, gis0),
    ]
    for h in idx_cp:
        h.wait()

    bufr = (bufr0, bufr1)
    bufi = (bufi0, bufi1)
    grs = (grs0, grs1)
    gis = (gis0, gis1)
    ors = (ors0, ors1)
    ois = (ois0, ois1)
    ghr = [None, None]
    ghi = [None, None]
    ohr = [None, None]
    ohi = [None, None]
    for c in range(NCHUNK):
        for j in range(CH // L):
            sl = pl.ds(c * CH + j * L, L)
            kv = ((pv[sl] * 8 + av[sl]) * 16 + ov[sl]) * 16 + gv[sl]
            kidx[c, pl.ds(j * L, L)] = kv
        slot = c % 2
        if ohr[slot] is not None:
            ohr[slot].wait()
            ohi[slot].wait()
        ghr[slot] = pltpu.async_copy(tr_h.at[kidx.at[c]], bufr[slot], grs[slot])
        ghi[slot] = pltpu.async_copy(ti_h.at[kidx.at[c]], bufi[slot], gis[slot])
        if c >= 1:
            prev = (c - 1) % 2
            dst = pl.ds(base + (c - 1) * CH, CH)
            ghr[prev].wait()
            ohr[prev] = pltpu.async_copy(bufr[prev], re_h.at[dst], ors[prev])
            ghi[prev].wait()
            ohi[prev] = pltpu.async_copy(bufi[prev], im_h.at[dst], ois[prev])
    last = (NCHUNK - 1) % 2
    dst = pl.ds(base + (NCHUNK - 1) * CH, CH)
    ghr[last].wait()
    ohr[last] = pltpu.async_copy(bufr[last], re_h.at[dst], ors[last])
    ghi[last].wait()
    ohi[last] = pltpu.async_copy(bufi[last], im_h.at[dst], ois[last])
    for h in aux_cp:
        h.wait()
    for c in range(NCHUNK):
        for j in range(CH // L):
            sl = pl.ds(c * CH + j * L, L)
            kv = kidx[c, pl.ds(j * L, L)]
            krow = lax.shift_right_logical(kv, 7)
            kcol = jnp.bitwise_and(kv, 127)
            salv[sl] = plsc.load_gather(sv, [krow, kcol]) + nzv[sl]
            cl = plsc.load_gather(cv, [krow, kcol])
            confv[sl] = 1.0 / (1.0 + jnp.exp(-cl))
    pltpu.sync_copy(salv, sal_h.at[pl.ds(base, BPW)])
    pltpu.sync_copy(confv, conf_h.at[pl.ds(base, BPW)])
    for h in ohr + ohi:
        if h is not None:
            h.wait()


def _sc_lookup(*args):
    return pl.kernel(
        _sc_body,
        out_type=[
            jax.ShapeDtypeStruct((B, DIM), jnp.float32),
            jax.ShapeDtypeStruct((B, DIM), jnp.float32),
            jax.ShapeDtypeStruct((B,), jnp.float32),
            jax.ShapeDtypeStruct((B,), jnp.float32),
        ],
        mesh=plsc.VectorSubcoreMesh(
            core_axis_name="c", subcore_axis_name="s",
            num_cores=NC, num_subcores=NS),
        compiler_params=pltpu.CompilerParams(needs_layout_passes=False),
        scratch_types=[
            pltpu.VMEM((NCHUNK, CH), jnp.int32),
            pltpu.VMEM((BPW,), jnp.int32),
            pltpu.VMEM((BPW,), jnp.int32),
            pltpu.VMEM((BPW,), jnp.int32),
            pltpu.VMEM((BPW,), jnp.int32),
            pltpu.VMEM((BPW,), jnp.float32),
            pltpu.VMEM((NCOMB // 128, 128), jnp.float32),
            pltpu.VMEM((NCOMB // 128, 128), jnp.float32),
            pltpu.VMEM((BPW,), jnp.float32),
            pltpu.VMEM((BPW,), jnp.float32),
            pltpu.VMEM((CH, DIM), jnp.float32),
            pltpu.VMEM((CH, DIM), jnp.float32),
            pltpu.VMEM((CH, DIM), jnp.float32),
            pltpu.VMEM((CH, DIM), jnp.float32),
            pltpu.SemaphoreType.DMA,
            pltpu.SemaphoreType.DMA,
            pltpu.SemaphoreType.DMA,
            pltpu.SemaphoreType.DMA,
            pltpu.SemaphoreType.DMA,
            pltpu.SemaphoreType.DMA,
            pltpu.SemaphoreType.DMA,
            pltpu.SemaphoreType.DMA,
        ],
    )(*args)


def kernel(p_idx, a_idx, o_idx, g_idx, noise, perspective_emb, audio_dir_emb,
           olfactory_loc_emb, gustatory_loc_emb, W_sal, b_sal, W_conf, b_conf):
    f32 = jnp.float32
    bs = b_sal.reshape(1, 1).astype(f32)
    bc = b_conf.reshape(1, 1).astype(f32)

    tr, ti, s, c = _build_tables(
        perspective_emb.astype(f32), audio_dir_emb.astype(f32),
        olfactory_loc_emb.astype(f32), gustatory_loc_emb.astype(f32),
        W_sal.astype(f32), W_conf.astype(f32), bs, bc)

    re, im, sal, conf = _sc_lookup(
        tr, ti, s, c,
        p_idx.astype(jnp.int32), a_idx.astype(jnp.int32),
        o_idx.astype(jnp.int32), g_idx.astype(jnp.int32),
        noise.astype(f32).reshape(B))

    proposal = lax.complex(re, im)
    return proposal, sal.reshape(B, 1), conf.reshape(B, 1)
